# baseline (speedup 1.0000x reference)
import numpy as np

import jax
import jax.numpy as jnp
from jax.experimental import pallas as pl
from jax.experimental.pallas import tpu as pltpu

_N = 100000
_ROWS = 512
_R = 8

_TINY = np.float32(np.finfo(np.float32).tiny)


def _threefry_bits(x1):
    k0 = jnp.uint32(0)
    k1 = jnp.uint32(42)
    k2 = k0 ^ k1 ^ jnp.uint32(0x1BD11BDA)
    ks = (k0, k1, k2)
    rot = ((13, 15, 26, 6), (17, 29, 16, 24))
    x0 = jnp.zeros_like(x1) + k0
    x1 = x1 + k1
    for grp in range(5):
        for r in rot[grp % 2]:
            x0 = x0 + x1
            x1 = (x1 << r) | (x1 >> (32 - r))
            x1 = x1 ^ x0
        x0 = x0 + ks[(grp + 1) % 3]
        x1 = x1 + ks[(grp + 2) % 3] + jnp.uint32(grp + 1)
    return x0 ^ x1


def _sample_kernel(p_ref, out_ref):
    i = pl.program_id(0)
    row0 = (i * _R).astype(jnp.uint32)
    rows = jax.lax.broadcasted_iota(jnp.uint32, (_R, _N), 0)
    cols = jax.lax.broadcasted_iota(jnp.uint32, (_R, _N), 1)
    ctr = (rows + row0) * jnp.uint32(_N) + cols
    bits = _threefry_bits(ctr)
    fb = (bits >> 9) | jnp.uint32(0x3F800000)
    f = jax.lax.bitcast_convert_type(fb, jnp.float32) - jnp.float32(1.0)
    u = jnp.maximum(_TINY, f * (jnp.float32(1.0) - _TINY) + _TINY)
    g = -jnp.log(-jnp.log(u))
    score = g + jnp.log(p_ref[...])
    m = jnp.max(score, axis=1, keepdims=True)
    icol = jax.lax.broadcasted_iota(jnp.int32, (_R, _N), 1)
    idx = jnp.min(jnp.where(score == m, icol, jnp.int32(_N)), axis=1,
                  keepdims=True)
    out_ref[...] = (icol == idx).astype(jnp.float32)


def kernel(input):
    p2d = jnp.reshape(input, (_ROWS, _N))
    out = pl.pallas_call(
        _sample_kernel,
        grid=(_ROWS // _R,),
        in_specs=[pl.BlockSpec((_R, _N), lambda i: (i, 0))],
        out_specs=pl.BlockSpec((_R, _N), lambda i: (i, 0)),
        out_shape=jax.ShapeDtypeStruct((_ROWS, _N), jnp.float32),
        compiler_params=pltpu.CompilerParams(
            dimension_semantics=("parallel",),
        ),
    )(p2d)
    return jnp.reshape(out, input.shape)
